# baseline (speedup 1.0000x reference)
import jax
import jax.numpy as jnp
from jax import lax
from jax.experimental import pallas as pl

PRE_NMS_THRESH = 0.05
PRE_NMS_TOP_N = 1000
NMS_THRESH = 0.6
FPN_POST_NMS_TOP_N = 100


def _score_body(cls_ref, ctr_ref, out_ref):
    cls = cls_ref[...]
    score = cls * ctr_ref[...]
    out_ref[...] = jnp.where(cls > PRE_NMS_THRESH, score, -jnp.inf)


def _masked_scores(cls2, ctr2):
    return pl.pallas_call(
        _score_body,
        out_shape=jax.ShapeDtypeStruct(cls2.shape, jnp.float32),
    )(cls2, ctr2)


def kernel(locations, box_cls, box_regression, centerness, image_sizes):
    locs = locations[0]
    cls = box_cls[0, 0]
    reg = box_regression[0, 0]
    ctr = centerness[0, 0, 0]
    C, H, W = cls.shape
    HW = H * W

    cls2 = cls.reshape(C, HW).T
    reg2 = reg.reshape(4, HW).T
    ctr2 = ctr.reshape(HW, 1)

    masked = _masked_scores(cls2, ctr2).reshape(-1)
    n_cand = jnp.minimum(jnp.sum(cls2 > PRE_NMS_THRESH), PRE_NMS_TOP_N)
    top_s, top_i = lax.top_k(masked, PRE_NMS_TOP_N)
    valid = (jnp.arange(PRE_NMS_TOP_N) < n_cand) & jnp.isfinite(top_s)
    loc_idx = top_i // C
    labels = top_i % C + 1
    plocs = jnp.take(locs, loc_idx, axis=0)
    pregs = jnp.take(reg2, loc_idx, axis=0)
    x1 = plocs[:, 0] - pregs[:, 0]
    y1 = plocs[:, 1] - pregs[:, 1]
    x2 = plocs[:, 0] + pregs[:, 2]
    y2 = plocs[:, 1] + pregs[:, 3]
    h = image_sizes[0, 0].astype(jnp.float32)
    w = image_sizes[0, 1].astype(jnp.float32)
    wmax = jnp.maximum(w - 1.0, 0.0)
    hmax = jnp.maximum(h - 1.0, 0.0)
    x1 = jnp.clip(x1, 0.0, wmax)
    x2 = jnp.clip(x2, 0.0, wmax)
    y1 = jnp.clip(y1, 0.0, hmax)
    y2 = jnp.clip(y2, 0.0, hmax)
    boxes = jnp.stack([x1, y1, x2, y2], axis=1)
    safe = jnp.where(valid & (top_s > 0.0), top_s, 1.0)
    det_scores = jnp.where(valid, jnp.sqrt(safe), 0.0)

    off = labels.astype(jnp.float32) * 10000.0
    nb = boxes + off[:, None]
    s = jnp.where(valid, det_scores, -jnp.inf)
    order = jnp.argsort(-s)
    b = jnp.take(nb, order, axis=0)
    v = jnp.take(valid, order, axis=0)
    ss = jnp.take(s, order, axis=0)
    areas = (b[:, 2] - b[:, 0]) * (b[:, 3] - b[:, 1])
    n = b.shape[0]
    idxs = jnp.arange(n)

    def body(i, keep):
        xx1 = jnp.maximum(b[i, 0], b[:, 0])
        yy1 = jnp.maximum(b[i, 1], b[:, 1])
        xx2 = jnp.minimum(b[i, 2], b[:, 2])
        yy2 = jnp.minimum(b[i, 3], b[:, 3])
        inter = jnp.maximum(xx2 - xx1, 0.0) * jnp.maximum(yy2 - yy1, 0.0)
        iou = inter / (areas[i] + areas - inter + 1e-9)
        sup = (iou > NMS_THRESH) & (idxs > i) & keep[i]
        return keep & (~sup)

    keep = lax.fori_loop(0, n, body, v)
    fs = jnp.where(keep, ss, -jnp.inf)
    topv, topi = lax.top_k(fs, FPN_POST_NMS_TOP_N)
    db = jnp.take(jnp.take(boxes, order, axis=0), topi, axis=0)
    dl = jnp.take(jnp.take(labels, order, axis=0), topi, axis=0)
    dv = jnp.isfinite(topv)
    ds = jnp.where(dv, topv, 0.0)
    return db, ds, dl


# speedup vs baseline: 3.4024x; 3.4024x over previous
import jax
import jax.numpy as jnp
from jax import lax
from jax.experimental import pallas as pl

PRE_NMS_THRESH = 0.05
PRE_NMS_TOP_N = 1000
NMS_THRESH = 0.6
FPN_POST_NMS_TOP_N = 100

N_PAD = 1024
OUT_PAD = 128


def _score_body(cls_ref, ctr_ref, out_ref):
    cls = cls_ref[...]
    score = cls * ctr_ref[...]
    out_ref[...] = jnp.where(cls > PRE_NMS_THRESH, score, -jnp.inf)


def _masked_scores(cls2, ctr2):
    return pl.pallas_call(
        _score_body,
        out_shape=jax.ShapeDtypeStruct(cls2.shape, jnp.float32),
    )(cls2, ctr2)


def _nms_body(sb_ref, sbt_ref, ss_col_ref, ss_row_ref, sl_col_ref,
              sl_row_ref, sv_col_ref, sv_row_ref,
              db_ref, ds_ref, dl_ref):
    n = N_PAD
    off_col = sl_col_ref[...] * 10000.0
    off_row = sl_row_ref[...] * 10000.0
    x1c = sb_ref[:, 0:1] + off_col
    y1c = sb_ref[:, 1:2] + off_col
    x2c = sb_ref[:, 2:3] + off_col
    y2c = sb_ref[:, 3:4] + off_col
    x1r = sbt_ref[0:1, :] + off_row
    y1r = sbt_ref[1:2, :] + off_row
    x2r = sbt_ref[2:3, :] + off_row
    y2r = sbt_ref[3:4, :] + off_row

    iw = jnp.maximum(jnp.minimum(x2c, x2r) - jnp.maximum(x1c, x1r), 0.0)
    ih = jnp.maximum(jnp.minimum(y2c, y2r) - jnp.maximum(y1c, y1r), 0.0)
    inter = iw * ih
    areas_col = (x2c - x1c) * (y2c - y1c)
    areas_row = (x2r - x1r) * (y2r - y1r)
    iou = inter / (areas_col + areas_row - inter + 1e-9)
    m = iou > NMS_THRESH

    ri = lax.broadcasted_iota(jnp.int32, (n, n), 0)
    ci = lax.broadcasted_iota(jnp.int32, (n, n), 1)
    striu = jnp.where(m & (ri < ci), 1.0, 0.0).astype(jnp.bfloat16)
    stril = jnp.where(m & (ri > ci), 1.0, 0.0).astype(jnp.bfloat16)

    v_col = sv_col_ref[...]
    v_row = sv_row_ref[...]

    def cond(c):
        _, _, changed = c
        return changed

    def body(c):
        k_col, k_row, _ = c
        sup_r = lax.dot_general(
            k_row.astype(jnp.bfloat16), striu,
            dimension_numbers=(((1,), (0,)), ((), ())),
            preferred_element_type=jnp.float32) > 0.0
        sup_c = lax.dot_general(
            stril, k_col.astype(jnp.bfloat16),
            dimension_numbers=(((1,), (0,)), ((), ())),
            preferred_element_type=jnp.float32) > 0.0
        nk_row = jnp.where(sup_r, 0.0, v_row)
        nk_col = jnp.where(sup_c, 0.0, v_col)
        changed = jnp.any(nk_col != k_col)
        return nk_col, nk_row, changed

    k_col, k_row, _ = lax.while_loop(
        cond, body, (v_col, v_row, jnp.bool_(True)))

    neg_inf = jnp.float32(-jnp.inf)
    fs_col = jnp.where(k_col > 0.0, ss_col_ref[...], neg_inf)
    fs_row = jnp.where(k_row > 0.0, ss_row_ref[...], neg_inf)

    higher = (fs_col > fs_row) | ((fs_col == fs_row) & (ri < ci))
    rank_row = jnp.sum(higher.astype(jnp.float32), axis=0, keepdims=True)

    r_col = lax.broadcasted_iota(jnp.int32, (OUT_PAD, n), 0).astype(jnp.float32)
    g = jnp.where(rank_row == r_col, 1.0, 0.0)

    gsel = g > 0.0
    topv = jnp.sum(jnp.where(gsel, fs_row, 0.0), axis=1,
                   keepdims=True)
    finite = jnp.isfinite(topv)
    ds_ref[...] = jnp.where(finite, topv, 0.0)
    dl_ref[...] = jnp.sum(g * sl_row_ref[...], axis=1, keepdims=True)
    for c in range(4):
        db_ref[:, c:c + 1] = jnp.sum(g * sbt_ref[c:c + 1, :], axis=1,
                                     keepdims=True)


def _nms_topk(sb, sbt, ss, sl, sv):
    n = N_PAD
    out = pl.pallas_call(
        _nms_body,
        out_shape=(
            jax.ShapeDtypeStruct((OUT_PAD, 4), jnp.float32),
            jax.ShapeDtypeStruct((OUT_PAD, 1), jnp.float32),
            jax.ShapeDtypeStruct((OUT_PAD, 1), jnp.float32),
        ),
    )(sb, sbt, ss.reshape(n, 1), ss.reshape(1, n), sl.reshape(n, 1),
      sl.reshape(1, n), sv.reshape(n, 1), sv.reshape(1, n))
    return out


def kernel(locations, box_cls, box_regression, centerness, image_sizes):
    locs = locations[0]
    cls = box_cls[0, 0]
    reg = box_regression[0, 0]
    ctr = centerness[0, 0, 0]
    C, H, W = cls.shape
    HW = H * W

    cls2 = cls.reshape(C, HW).T
    reg2 = reg.reshape(4, HW).T
    ctr2 = ctr.reshape(HW, 1)

    masked = _masked_scores(cls2, ctr2).reshape(-1)
    n_cand = jnp.minimum(jnp.sum(cls2 > PRE_NMS_THRESH), PRE_NMS_TOP_N)
    top_s, top_i = lax.top_k(masked, PRE_NMS_TOP_N)
    valid = (jnp.arange(PRE_NMS_TOP_N) < n_cand) & jnp.isfinite(top_s)
    loc_idx = top_i // C
    labels = top_i % C + 1
    plocs = jnp.take(locs, loc_idx, axis=0)
    pregs = jnp.take(reg2, loc_idx, axis=0)
    x1 = plocs[:, 0] - pregs[:, 0]
    y1 = plocs[:, 1] - pregs[:, 1]
    x2 = plocs[:, 0] + pregs[:, 2]
    y2 = plocs[:, 1] + pregs[:, 3]
    h = image_sizes[0, 0].astype(jnp.float32)
    w = image_sizes[0, 1].astype(jnp.float32)
    wmax = jnp.maximum(w - 1.0, 0.0)
    hmax = jnp.maximum(h - 1.0, 0.0)
    x1 = jnp.clip(x1, 0.0, wmax)
    x2 = jnp.clip(x2, 0.0, wmax)
    y1 = jnp.clip(y1, 0.0, hmax)
    y2 = jnp.clip(y2, 0.0, hmax)
    boxes = jnp.stack([x1, y1, x2, y2], axis=1)
    safe = jnp.where(valid & (top_s > 0.0), top_s, 1.0)
    det_scores = jnp.where(valid, jnp.sqrt(safe), 0.0)

    s = jnp.where(valid, det_scores, -jnp.inf)
    order = jnp.argsort(-s)
    sb = jnp.take(boxes, order, axis=0)
    sv = jnp.take(valid, order, axis=0).astype(jnp.float32)
    ss = jnp.take(s, order, axis=0)
    sl = jnp.take(labels, order, axis=0).astype(jnp.float32)

    pad = N_PAD - PRE_NMS_TOP_N
    sb = jnp.concatenate([sb, jnp.zeros((pad, 4), jnp.float32)], axis=0)
    sv = jnp.concatenate([sv, jnp.zeros((pad,), jnp.float32)], axis=0)
    ss = jnp.concatenate([ss, jnp.full((pad,), -jnp.inf, jnp.float32)],
                         axis=0)
    sl = jnp.concatenate([sl, jnp.zeros((pad,), jnp.float32)], axis=0)

    db_p, ds_p, dl_p = _nms_topk(sb, sb.T, ss, sl, sv)
    db = db_p[:FPN_POST_NMS_TOP_N]
    ds = ds_p[:FPN_POST_NMS_TOP_N, 0]
    dl = dl_p[:FPN_POST_NMS_TOP_N, 0].astype(jnp.int32)
    return db, ds, dl
